# baseline (speedup 1.0000x reference)
import jax
import jax.numpy as jnp
from jax import lax
from jax.experimental import pallas as pl
from jax.experimental.pallas import tpu as pltpu
from jax.experimental.pallas import tpu_sc as plsc

_NC = 2
_NS = 16
_NW = _NC * _NS
_CHUNK = 256
_LANES = 16


def kernel(token_ids, W):
    B, L = token_ids.shape
    n = B * L
    V, dim = W.shape
    per_line = 128 // dim
    w4 = W.reshape(V // per_line, 128)
    idx = token_ids.reshape(n)

    b_per_w = n // _NW
    n_chunks = b_per_w // _CHUNK

    mesh = plsc.VectorSubcoreMesh(core_axis_name="c", subcore_axis_name="s")

    @pl.kernel(
        out_type=jax.ShapeDtypeStruct((n, dim), W.dtype),
        mesh=mesh,
        compiler_params=pltpu.CompilerParams(needs_layout_passes=False),
        scratch_types=[
            pltpu.VMEM((_CHUNK,), jnp.int32),
            pltpu.VMEM((_CHUNK,), jnp.int32),
            pltpu.VMEM((_CHUNK,), jnp.int32),
            pltpu.VMEM((_CHUNK, 128), jnp.float32),
            pltpu.VMEM((_CHUNK, dim), jnp.float32),
            pltpu.SemaphoreType.DMA,
        ],
    )
    def gather_kernel(w_hbm, i_hbm, o_hbm, idx_v, q_v, r_v, wide_v, out_v, sem):
        wid = lax.axis_index("s") * _NC + lax.axis_index("c")
        base = wid * b_per_w
        lanes = lax.iota(jnp.int32, _LANES)

        @pl.loop(0, n_chunks)
        def _(j):
            off = base + j * _CHUNK
            pltpu.sync_copy(i_hbm.at[pl.ds(off, _CHUNK)], idx_v)

            @pl.loop(0, _CHUNK, step=_LANES)
            def _(c):
                v = idx_v.at[pl.ds(c, _LANES)][...]
                q_v.at[pl.ds(c, _LANES)][...] = v >> 2
                r_v.at[pl.ds(c, _LANES)][...] = (v & 3) * dim

            pltpu.async_copy(w_hbm.at[q_v], wide_v, sem).wait()

            @pl.loop(0, _CHUNK, step=_LANES)
            def _(g):
                row16 = g + lanes
                col0 = r_v.at[pl.ds(g, _LANES)][...]
                for k in range(dim):
                    val = plsc.load_gather(wide_v, [row16, col0 + k])
                    plsc.store_scatter(out_v, [row16, jnp.full((_LANES,), k, jnp.int32)], val)

            pltpu.sync_copy(out_v, o_hbm.at[pl.ds(off, _CHUNK)])

    out = gather_kernel(w4, idx)
    return out.reshape(B, L, dim)
